# baseline (speedup 1.0000x reference)
import functools

import jax
import jax.numpy as jnp
from jax import lax
from jax.experimental import pallas as pl
from jax.experimental.pallas import tpu as pltpu
from jax.experimental.pallas import tpu_sc as plsc

_LANES_PER_TILE = 128


def kernel(Xnew, mu):
    B, F = Xnew.shape
    V = mu.shape[0]
    info = plsc.get_sparse_core_info()
    NC, NS, L = info.num_cores, info.num_subcores, info.num_lanes
    NW = NC * NS
    bpw = B // NW
    T = B // _LANES_PER_TILE
    tpw = T // NW

    mesh = plsc.VectorSubcoreMesh(
        core_axis_name="c", subcore_axis_name="s", num_cores=NC
    )

    @functools.partial(
        pl.kernel,
        mesh=mesh,
        compiler_params=pltpu.CompilerParams(
            needs_layout_passes=False,
            use_tc_tiling_on_sc=False,
            skip_device_barrier=True,
            disable_bounds_checks=True,
            disable_semaphore_checks=True,
        ),
        out_type=[
            jax.ShapeDtypeStruct((B,), jnp.float32),
            jax.ShapeDtypeStruct((B,), jnp.float32),
        ],
        scratch_types=[
            pltpu.VMEM((V,), jnp.float32),
            pltpu.VMEM((tpw, 1, _LANES_PER_TILE), jnp.float32),
            pltpu.VMEM((bpw,), jnp.float32),
            pltpu.VMEM((bpw,), jnp.float32),
            pltpu.SemaphoreType.DMA,
            pltpu.SemaphoreType.DMA,
        ],
    )
    def sc_lookup(y_hbm, mu_hbm, m_hbm, v_hbm, mu_v, col_v, m_v, var_v, s0, s1):
        wid = lax.axis_index("s") * NC + lax.axis_index("c")
        base = wid * bpw
        cp0 = pltpu.async_copy(mu_hbm, mu_v, s0)
        cp1 = pltpu.async_copy(
            y_hbm.at[pl.ds(wid * tpw, tpw), pl.ds(F - 1, 1)], col_v, s1
        )
        cp0.wait()
        cp1.wait()

        def body(tt, carry):
            for c in range(_LANES_PER_TILE // L):
                o = tt * _LANES_PER_TILE + c * L
                catf = col_v[tt, 0, pl.ds(c * L, L)]
                cat = catf.astype(jnp.int32)
                m = plsc.load_gather(mu_v, [cat])
                m_v[pl.ds(o, L)] = m
                var_v[pl.ds(o, L)] = m - m * m
            return carry

        lax.fori_loop(0, tpw, body, 0)

        pltpu.sync_copy(m_v, m_hbm.at[pl.ds(base, bpw)])
        pltpu.sync_copy(var_v, v_hbm.at[pl.ds(base, bpw)])

    Y = Xnew.reshape(T, _LANES_PER_TILE, F).transpose(0, 2, 1)
    m, var = sc_lookup(Y, mu)
    return (m.reshape(B, 1), var.reshape(B, 1))


# speedup vs baseline: 1.0620x; 1.0620x over previous
import functools

import jax
import jax.numpy as jnp
from jax import lax
from jax.experimental import pallas as pl
from jax.experimental.pallas import tpu as pltpu
from jax.experimental.pallas import tpu_sc as plsc

_LANES_PER_TILE = 128


def kernel(Xnew, mu):
    B, F = Xnew.shape
    V = mu.shape[0]
    info = plsc.get_sparse_core_info()
    NC, NS, L = 1, info.num_subcores, info.num_lanes
    NW = NC * NS
    bpw = B // NW
    T = B // _LANES_PER_TILE
    tpw = T // NW

    mesh = plsc.VectorSubcoreMesh(
        core_axis_name="c", subcore_axis_name="s", num_cores=NC
    )

    @functools.partial(
        pl.kernel,
        mesh=mesh,
        compiler_params=pltpu.CompilerParams(
            needs_layout_passes=False,
            use_tc_tiling_on_sc=False,
            skip_device_barrier=True,
            disable_bounds_checks=True,
            disable_semaphore_checks=True,
        ),
        out_type=[
            jax.ShapeDtypeStruct((B,), jnp.float32),
            jax.ShapeDtypeStruct((B,), jnp.float32),
        ],
        scratch_types=[
            pltpu.VMEM((V,), jnp.float32),
            pltpu.VMEM((tpw, 1, _LANES_PER_TILE), jnp.float32),
            pltpu.VMEM((bpw,), jnp.float32),
            pltpu.VMEM((bpw,), jnp.float32),
            pltpu.SemaphoreType.DMA,
            pltpu.SemaphoreType.DMA,
        ],
    )
    def sc_lookup(y_hbm, mu_hbm, m_hbm, v_hbm, mu_v, col_v, m_v, var_v, s0, s1):
        wid = lax.axis_index("s") * NC + lax.axis_index("c")
        base = wid * bpw
        cp0 = pltpu.async_copy(mu_hbm, mu_v, s0)
        cp1 = pltpu.async_copy(
            y_hbm.at[pl.ds(wid * tpw, tpw), pl.ds(F - 1, 1)], col_v, s1
        )
        cp0.wait()
        cp1.wait()

        cpl = _LANES_PER_TILE // L

        def body(j, carry):
            tt = j // cpl
            c = j - tt * cpl
            catf = col_v[tt, 0, pl.ds(c * L, L)]
            cat = catf.astype(jnp.int32)
            m = plsc.load_gather(mu_v, [cat])
            m_v[pl.ds(j * L, L)] = m
            var_v[pl.ds(j * L, L)] = m - m * m
            return carry

        lax.fori_loop(0, bpw // L, body, 0)

        pltpu.sync_copy(m_v, m_hbm.at[pl.ds(base, bpw)])
        pltpu.sync_copy(var_v, v_hbm.at[pl.ds(base, bpw)])

    Y = Xnew.reshape(T, _LANES_PER_TILE, F).transpose(0, 2, 1)
    m, var = sc_lookup(Y, mu)
    return (m.reshape(B, 1), var.reshape(B, 1))


# speedup vs baseline: 1.0785x; 1.0155x over previous
import functools

import jax
import jax.numpy as jnp
from jax import lax
from jax.experimental import pallas as pl
from jax.experimental.pallas import tpu as pltpu
from jax.experimental.pallas import tpu_sc as plsc

_LANES_PER_TILE = 128


def kernel(Xnew, mu):
    B, F = Xnew.shape
    V = mu.shape[0]
    info = plsc.get_sparse_core_info()
    NC, NS, L = 1, info.num_subcores, info.num_lanes
    NW = NC * NS
    bpw = B // NW
    T = B // _LANES_PER_TILE
    tpw = T // NW

    mesh = plsc.VectorSubcoreMesh(
        core_axis_name="c", subcore_axis_name="s", num_cores=NC
    )

    @functools.partial(
        pl.kernel,
        mesh=mesh,
        compiler_params=pltpu.CompilerParams(
            needs_layout_passes=False,
            use_tc_tiling_on_sc=False,
            skip_device_barrier=True,
            disable_bounds_checks=True,
            disable_semaphore_checks=True,
        ),
        out_type=[
            jax.ShapeDtypeStruct((B,), jnp.float32),
            jax.ShapeDtypeStruct((B,), jnp.float32),
        ],
        scratch_types=[
            pltpu.VMEM((V,), jnp.float32),
            pltpu.VMEM((tpw, 1, _LANES_PER_TILE), jnp.float32),
            pltpu.VMEM((bpw,), jnp.float32),
            pltpu.VMEM((bpw,), jnp.float32),
            pltpu.SemaphoreType.DMA,
            pltpu.SemaphoreType.DMA,
        ],
    )
    def sc_lookup(y_hbm, mu_hbm, m_hbm, v_hbm, mu_v, col_v, m_v, var_v, s0, s1):
        wid = lax.axis_index("s") * NC + lax.axis_index("c")
        base = wid * bpw
        cp0 = pltpu.async_copy(mu_hbm, mu_v, s0)
        cp1 = pltpu.async_copy(
            y_hbm.at[pl.ds(wid * tpw, tpw), pl.ds(F - 1, 1)], col_v, s1
        )
        cp0.wait()
        cp1.wait()

        def body(tt, carry):
            for c in range(_LANES_PER_TILE // L):
                o = tt * _LANES_PER_TILE + c * L
                catf = col_v[tt, 0, pl.ds(c * L, L)]
                cat = catf.astype(jnp.int32)
                m = plsc.load_gather(mu_v, [cat])
                m_v[pl.ds(o, L)] = m
                var_v[pl.ds(o, L)] = m - m * m
            return carry

        lax.fori_loop(0, tpw, body, 0)

        pltpu.sync_copy(m_v, m_hbm.at[pl.ds(base, bpw)])
        pltpu.sync_copy(var_v, v_hbm.at[pl.ds(base, bpw)])

    Y = Xnew.reshape(T, _LANES_PER_TILE, F).transpose(0, 2, 1)
    m, var = sc_lookup(Y, mu)
    return (m.reshape(B, 1), var.reshape(B, 1))


# speedup vs baseline: 1.0803x; 1.0017x over previous
import functools

import jax
import jax.numpy as jnp
from jax import lax
from jax.experimental import pallas as pl
from jax.experimental.pallas import tpu as pltpu
from jax.experimental.pallas import tpu_sc as plsc

_LANES_PER_TILE = 128


def kernel(Xnew, mu):
    B, F = Xnew.shape
    V = mu.shape[0]
    info = plsc.get_sparse_core_info()
    NC, NS, L = 1, info.num_subcores, info.num_lanes
    NW = NC * NS
    bpw = B // NW
    T = B // _LANES_PER_TILE
    tpw = T // NW

    mesh = plsc.VectorSubcoreMesh(
        core_axis_name="c", subcore_axis_name="s", num_cores=NC
    )

    @functools.partial(
        pl.kernel,
        mesh=mesh,
        compiler_params=pltpu.CompilerParams(
            needs_layout_passes=False,
            use_tc_tiling_on_sc=False,
            skip_device_barrier=True,
            disable_bounds_checks=True,
            disable_semaphore_checks=True,
        ),
        out_type=[
            jax.ShapeDtypeStruct((B,), jnp.float32),
            jax.ShapeDtypeStruct((B,), jnp.float32),
        ],
        scratch_types=[
            pltpu.VMEM((V,), jnp.float32),
            pltpu.VMEM((tpw, 1, _LANES_PER_TILE), jnp.float32),
            pltpu.VMEM((bpw,), jnp.float32),
            pltpu.VMEM((bpw,), jnp.float32),
            pltpu.SemaphoreType.DMA,
            pltpu.SemaphoreType.DMA,
        ],
    )
    def sc_lookup(y_hbm, mu_hbm, m_hbm, v_hbm, mu_v, col_v, m_v, var_v, s0, s1):
        wid = lax.axis_index("s") * NC + lax.axis_index("c")
        base = wid * bpw
        cp0 = pltpu.async_copy(mu_hbm, mu_v, s0)
        cp1 = pltpu.async_copy(
            y_hbm.at[pl.ds(wid * tpw, tpw), pl.ds(F - 1, 1)], col_v, s1
        )
        cp0.wait()
        cp1.wait()

        def body(tt, carry):
            for c in range(_LANES_PER_TILE // L):
                o = tt * _LANES_PER_TILE + c * L
                catf = col_v[tt, 0, pl.ds(c * L, L)]
                cat = catf.astype(jnp.int32)
                m = plsc.load_gather(mu_v, [cat])
                m_v[pl.ds(o, L)] = m
                var_v[pl.ds(o, L)] = m - m * m
            return carry

        lax.fori_loop(0, tpw, body, 0)

        cp2 = pltpu.async_copy(m_v, m_hbm.at[pl.ds(base, bpw)], s0)
        cp3 = pltpu.async_copy(var_v, v_hbm.at[pl.ds(base, bpw)], s1)
        cp2.wait()
        cp3.wait()

    Y = Xnew.reshape(T, _LANES_PER_TILE, F).transpose(0, 2, 1)
    m, var = sc_lookup(Y, mu)
    return (m.reshape(B, 1), var.reshape(B, 1))
